# baseline (speedup 1.0000x reference)
import functools

import jax
import jax.numpy as jnp
import numpy as np
from jax import lax
from jax.experimental import pallas as pl
from jax.experimental.pallas import tpu as pltpu
from jax.experimental.pallas import tpu_sc as plsc

_N = 10000
_NOUT = 80000
_NPAD = 81920
_M = 150000
_MPAD = 150016
_CIN = 128
_COUT = 64
_S2 = 128
_BLK = 1000
_NB = 80
_TAP_ROWS = 81000
_ZROW = _NOUT
_EPS = 1e-4

_OFF2_NP = np.array([[a, b, c] for a in (0, 1) for b in (0, 1) for c in (0, 1)], dtype=np.int32)
_OFF3_NP = np.array([[a, b, c] for a in (-1, 0, 1) for b in (-1, 0, 1) for c in (-1, 0, 1)], dtype=np.int32)

_E3_NP = np.zeros((8, 27), np.int32)
_KP_NP = np.zeros((8, 27), np.int32)
for _k in range(8):
    for _k3 in range(27):
        _s = _OFF2_NP[_k] + _OFF3_NP[_k3]
        _e = _s >> 1
        _op = _s & 1
        _E3_NP[_k, _k3] = (_e[0] + 1) * 9 + (_e[1] + 1) * 3 + (_e[2] + 1)
        _KP_NP[_k, _k3] = _op[0] * 4 + _op[1] * 2 + _op[2]

_MESH = plsc.VectorSubcoreMesh(core_axis_name="c", subcore_axis_name="s")


def _up_body(x_ref, w_ref, o_ref):
    o_ref[0] = jnp.dot(x_ref[...], w_ref[0], preferred_element_type=jnp.float32)


def _upsample(x, w_up):
    return pl.pallas_call(
        _up_body,
        grid=(8, 10),
        in_specs=[
            pl.BlockSpec((_BLK, _CIN), lambda k, j: (j, 0)),
            pl.BlockSpec((1, _CIN, _COUT), lambda k, j: (k, 0, 0)),
        ],
        out_specs=pl.BlockSpec((1, _BLK, _COUT), lambda k, j: (k, j, 0)),
        out_shape=jax.ShapeDtypeStruct((8, _N, _COUT), jnp.float32),
    )(x, w_up)


def _p1_body(f_ref, sg_ref, w_ref, o_ref):
    b = pl.program_id(0)

    @pl.when(b == _NB)
    def _():
        o_ref[...] = jnp.zeros((_BLK, _COUT), jnp.float32)

    @pl.when(b < _NB)
    def _():
        h = f_ref[...] + sg_ref[...]
        o_ref[...] = jnp.dot(h, w_ref[0], preferred_element_type=jnp.float32)


def _pmid_body(s_ref, st_ref, gb_ref, w_ref, o_ref):
    b = pl.program_id(0)

    @pl.when(b == _NB)
    def _():
        o_ref[...] = jnp.zeros((_BLK, _COUT), jnp.float32)

    @pl.when(b < _NB)
    def _():
        st = st_ref[...]
        gb = gb_ref[...]
        mu = st[0:1, :] / float(_NOUT)
        var = st[1:2, :] / float(_NOUT) - mu * mu
        a = gb[0:1, :] * lax.rsqrt(var + _EPS)
        c = gb[1:2, :] - mu * a
        h = s_ref[...] * a + c
        h = jnp.where(h >= 0, h, 0.05 * h)
        o_ref[...] = jnp.dot(h, w_ref[0], preferred_element_type=jnp.float32)


def _p_table_first(feat_pad, skipg, w):
    return pl.pallas_call(
        _p1_body,
        grid=(_NB + 1, 27),
        in_specs=[
            pl.BlockSpec((_BLK, _COUT), lambda b, k: (b, 0)),
            pl.BlockSpec((_BLK, _COUT), lambda b, k: (b, 0)),
            pl.BlockSpec((1, _COUT, _COUT), lambda b, k: (k, 0, 0)),
        ],
        out_specs=pl.BlockSpec((_BLK, _COUT), lambda b, k: (k * (_NB + 1) + b, 0)),
        out_shape=jax.ShapeDtypeStruct((27 * _TAP_ROWS, _COUT), jnp.float32),
    )(feat_pad, skipg, w)


def _p_table_mid(s_prev, stats, gb, w):
    return pl.pallas_call(
        _pmid_body,
        grid=(_NB + 1, 27),
        in_specs=[
            pl.BlockSpec((_BLK, _COUT), lambda b, k: (b, 0)),
            pl.BlockSpec((8, _COUT), lambda b, k: (0, 0)),
            pl.BlockSpec((8, _COUT), lambda b, k: (0, 0)),
            pl.BlockSpec((1, _COUT, _COUT), lambda b, k: (k, 0, 0)),
        ],
        out_specs=pl.BlockSpec((_BLK, _COUT), lambda b, k: (k * (_NB + 1) + b, 0)),
        out_shape=jax.ShapeDtypeStruct((27 * _TAP_ROWS, _COUT), jnp.float32),
    )(s_prev, stats, gb, w)


def _stats_body(s_ref, o_ref):
    i = pl.program_id(0)

    @pl.when(i == 0)
    def _():
        o_ref[...] = jnp.zeros((8, _COUT), jnp.float32)

    x = s_ref[...]
    o_ref[0:1, :] += jnp.sum(x, axis=0, keepdims=True)
    o_ref[1:2, :] += jnp.sum(x * x, axis=0, keepdims=True)


def _stats(s):
    return pl.pallas_call(
        _stats_body,
        grid=(625,),
        in_specs=[pl.BlockSpec((128, _COUT), lambda i: (i, 0))],
        out_specs=pl.BlockSpec((8, _COUT), lambda i: (0, 0)),
        out_shape=jax.ShapeDtypeStruct((8, _COUT), jnp.float32),
    )(s)


def _final_body(s1_ref, s3_ref, st1_ref, st3_ref, gb_ref, o_ref):
    st1 = st1_ref[...]
    st3 = st3_ref[...]
    gb = gb_ref[...]
    mu1 = st1[0:1, :] / float(_NOUT)
    var1 = st1[1:2, :] / float(_NOUT) - mu1 * mu1
    a1 = gb[0:1, :] * lax.rsqrt(var1 + _EPS)
    c1 = gb[1:2, :] - mu1 * a1
    h1 = s1_ref[...] * a1 + c1
    h1 = jnp.where(h1 >= 0, h1, 0.05 * h1)
    mu3 = st3[0:1, :] / float(_NOUT)
    var3 = st3[1:2, :] / float(_NOUT) - mu3 * mu3
    a3 = gb[2:3, :] * lax.rsqrt(var3 + _EPS)
    c3 = gb[3:4, :] - mu3 * a3
    s = s3_ref[...] * a3 + c3 + h1
    o_ref[...] = jnp.where(s >= 0, s, 0.333 * s)


def _final(s1, s3, st1, st3, gbf):
    return pl.pallas_call(
        _final_body,
        grid=(_NB,),
        in_specs=[
            pl.BlockSpec((_BLK, _COUT), lambda i: (i, 0)),
            pl.BlockSpec((_BLK, _COUT), lambda i: (i, 0)),
            pl.BlockSpec((8, _COUT), lambda i: (0, 0)),
            pl.BlockSpec((8, _COUT), lambda i: (0, 0)),
            pl.BlockSpec((8, _COUT), lambda i: (0, 0)),
        ],
        out_specs=pl.BlockSpec((_BLK, _COUT), lambda i: (i, 0)),
        out_shape=jax.ShapeDtypeStruct((_NOUT, _COUT), jnp.float32),
    )(s1, s3, st1, st3, gbf)


@functools.partial(
    pl.kernel,
    mesh=_MESH,
    out_type=jax.ShapeDtypeStruct((_NPAD, _COUT), jnp.float32),
    scratch_types=[
        pltpu.VMEM((128,), jnp.int32),
        pltpu.VMEM((128, _COUT), jnp.float32),
        pltpu.SemaphoreType.DMA,
    ],
    compiler_params=pltpu.CompilerParams(use_tc_tiling_on_sc=False),
)
def _sc_skip_gather(skip_hbm, sr_hbm, out_hbm, idx_v, buf_v, sem):
    wid = lax.axis_index("s") * 2 + lax.axis_index("c")

    def chunk(c, carry):
        ch = wid * 20 + c
        base = ch * 128
        pltpu.sync_copy(sr_hbm.at[pl.ds(base, 128)], idx_v)
        pltpu.async_copy(skip_hbm.at[idx_v], buf_v, sem).wait()
        pltpu.sync_copy(buf_v, out_hbm.at[pl.ds(base, 128)])
        return carry

    lax.fori_loop(0, 20, chunk, 0)


@functools.partial(
    pl.kernel,
    mesh=_MESH,
    out_type=jax.ShapeDtypeStruct((_NPAD, _COUT), jnp.float32),
    scratch_types=[
        pltpu.VMEM((512,), jnp.int32),
        pltpu.VMEM((512,), jnp.int32),
        pltpu.VMEM((512, _COUT), jnp.float32),
        pltpu.VMEM((512, _COUT), jnp.float32),
        pltpu.VMEM((16, _COUT), jnp.float32),
        pltpu.SemaphoreType.DMA,
        pltpu.SemaphoreType.DMA,
        pltpu.SemaphoreType.DMA,
        pltpu.SemaphoreType.DMA,
    ],
    compiler_params=pltpu.CompilerParams(use_tc_tiling_on_sc=False),
)
def _sc_gather_sum(p_hbm, cidx_hbm, out_hbm, idx0, idx1, buf0, buf1, acc_v,
                   isem0, isem1, gsem0, gsem1):
    wid = lax.axis_index("s") * 2 + lax.axis_index("c")
    g0 = wid * 160
    idx = (idx0, idx1)
    buf = (buf0, buf1)
    isem = (isem0, isem1)
    gsem = (gsem0, gsem1)

    def fire_idx(cg, slot):
        pltpu.async_copy(cidx_hbm.at[pl.ds(cg * 512, 512)], idx[slot], isem[slot])

    def drain_idx(slot):
        pltpu.make_async_copy(
            cidx_hbm.at[pl.ds(0, 512)], idx[slot], isem[slot]).wait()

    def fire_gather(slot):
        for j in range(4):
            pltpu.async_copy(
                p_hbm.at[idx[slot].at[pl.ds(j * 128, 128)]],
                buf[slot].at[pl.ds(j * 128, 128)],
                gsem[slot],
            )

    def drain_gather(slot):
        pltpu.make_async_copy(
            p_hbm.at[pl.ds(0, 512)], buf[slot], gsem[slot]).wait()

    fire_idx(g0, 0)
    drain_idx(0)
    fire_gather(0)
    fire_idx(g0 + 1, 1)

    def step(cp, carry):
        for b in (0, 1):
            c = cp * 2 + b
            other = 1 - b
            drain_idx(other)
            fire_gather(other)
            drain_gather(b)
            fire_idx(g0 + c + 2, b)
            bv = buf[b]
            for r in range(16):
                for l in range(4):
                    acc_v[r, pl.ds(16 * l, 16)] = bv[r * 27, pl.ds(16 * l, 16)]

            def tapk(k, cr):
                for r in range(16):
                    base = r * 27 + k
                    for l in range(4):
                        plsc.addupdate(
                            acc_v.at[r, pl.ds(16 * l, 16)],
                            bv[base, pl.ds(16 * l, 16)],
                        )
                return cr

            lax.fori_loop(1, 27, tapk, 0)
            pltpu.sync_copy(acc_v, out_hbm.at[pl.ds((g0 + c) * 16, 16)])
        return carry

    lax.fori_loop(0, 80, step, 0)
    drain_gather(0)
    drain_idx(1)


def _lin(c, s):
    return (c[:, 0] * s + c[:, 1]) * s + c[:, 2]


def kernel(x, skip_features, cords, skip_cords, W_up, W1, g1, b1, W2, g2, b2,
           W3, g3, b3, spatial_size):
    s_t = 2 * spatial_size
    out_coords = (2 * cords[None, :, :] + jnp.asarray(_OFF2_NP)[:, None, :]).reshape(-1, 3)

    id_x = _lin(out_coords, s_t)
    id_s = _lin(skip_cords, s_t)
    nvox = _S2 * _S2 * _S2
    lut_s = jnp.full((nvox,), -1, jnp.int32).at[id_s].set(
        jnp.arange(_M, dtype=jnp.int32))
    r = lut_s[id_x]
    sr = jnp.where(r >= 0, r, _M)
    sr_pad = jnp.full((_NPAD,), _M, jnp.int32).at[:_NOUT].set(sr)

    pid = (cords[:, 0] * 64 + cords[:, 1]) * 64 + cords[:, 2]
    plut = jnp.full((262144,), -1, jnp.int32).at[pid].set(
        jnp.arange(_N, dtype=jnp.int32))
    q = cords[None, :, :] + jnp.asarray(_OFF3_NP)[:, None, :]
    vq = jnp.all((q >= 0) & (q < 64), axis=-1)
    qid = jnp.clip((q[..., 0] * 64 + q[..., 1]) * 64 + q[..., 2], 0, 262143)
    pr = jnp.where(vq, plut[qid], -1)
    prn = pr[jnp.asarray(_E3_NP)]
    k3arr = jnp.arange(27, dtype=jnp.int32)[None, :, None]
    rows = jnp.asarray(_KP_NP)[:, :, None] * _N + prn
    cidx = jnp.where(prn >= 0, k3arr * _TAP_ROWS + rows, _ZROW)
    cidx = cidx.transpose(1, 0, 2).reshape(27, _NOUT)
    cidx_t = jnp.full((_NPAD, 27), _ZROW, jnp.int32).at[:_NOUT].set(cidx.T)
    cidx_c = cidx_t.reshape(5120, 16 * 27)
    cidx_pad = jnp.concatenate(
        [cidx_c, jnp.full((5120, 80), _ZROW, jnp.int32)], axis=1)
    cidx_pad = jnp.concatenate(
        [cidx_pad, jnp.full((2, 512), _ZROW, jnp.int32)], axis=0).reshape(-1)

    skip_ext = jnp.zeros((_MPAD, _COUT), jnp.float32).at[:_M].set(skip_features)
    gb1 = jnp.zeros((8, _COUT), jnp.float32).at[0].set(g1).at[1].set(b1)
    gb2 = jnp.zeros((8, _COUT), jnp.float32).at[0].set(g2).at[1].set(b2)
    gbf = (jnp.zeros((8, _COUT), jnp.float32)
           .at[0].set(g1).at[1].set(b1).at[2].set(g3).at[3].set(b3))

    skipg = _sc_skip_gather(skip_ext, sr_pad)
    feat = _upsample(x, W_up).reshape(_NOUT, _COUT)
    feat_pad = jnp.zeros((_NPAD, _COUT), jnp.float32).at[:_NOUT].set(feat)

    p1 = _p_table_first(feat_pad, skipg, W1)
    s1 = _sc_gather_sum(p1, cidx_pad)
    st1 = _stats(s1)

    p2 = _p_table_mid(s1, st1, gb1, W2)
    s2 = _sc_gather_sum(p2, cidx_pad)
    st2 = _stats(s2)

    p3 = _p_table_mid(s2, st2, gb2, W3)
    s3 = _sc_gather_sum(p3, cidx_pad)
    st3 = _stats(s3)

    return _final(s1, s3, st1, st3, gbf)
